# baseline (speedup 1.0000x reference)
import functools

import jax
import jax.numpy as jnp
from jax import lax
from jax.experimental import pallas as pl
from jax.experimental.pallas import tpu as pltpu
from jax.experimental.pallas import tpu_sc as plsc

VOC = 1000000
D = 32
B = 16384
NW = 32
BPW = B // NW
NCH = 4
CH = BPW // NCH


def _sc_pred_kernel():
    mesh = plsc.VectorSubcoreMesh(core_axis_name="c", subcore_axis_name="s")

    @functools.partial(
        pl.kernel,
        mesh=mesh,
        out_type=jax.ShapeDtypeStruct((B,), jnp.float32),
        compiler_params=pltpu.CompilerParams(
            needs_layout_passes=False, use_tc_tiling_on_sc=False),
        scratch_types=[
            pltpu.VMEM((NCH, CH), jnp.int32),
            pltpu.VMEM((NCH, CH), jnp.int32),
            pltpu.VMEM((BPW, D), jnp.float32),
            pltpu.VMEM((BPW, D), jnp.float32),
            pltpu.VMEM((BPW,), jnp.float32),
            pltpu.VMEM((BPW,), jnp.float32),
            pltpu.VMEM((16, 16), jnp.float32),
            pltpu.VMEM((BPW,), jnp.float32),
            pltpu.SemaphoreType.DMA,
        ],
    )
    def k(center_h, outside_h, wc_h, wo_h, bc_h, bo_h, pred_h,
          idxc, idxo, cbuf, obuf, bcv, bov, sbuf, predv, sem):
        wid = lax.axis_index("c") * 16 + lax.axis_index("s")

        pltpu.sync_copy(center_h.at[wid], idxc)
        pltpu.sync_copy(outside_h.at[wid], idxo)

        copies = []
        for ch in range(NCH):
            sl = pl.ds(ch * CH, CH)
            copies.append(pltpu.async_copy(
                wc_h.at[idxc.at[ch]], cbuf.at[sl, :], sem))
            copies.append(pltpu.async_copy(
                wo_h.at[idxo.at[ch]], obuf.at[sl, :], sem))
            copies.append(pltpu.async_copy(
                bc_h.at[idxc.at[ch]], bcv.at[sl], sem))
            copies.append(pltpu.async_copy(
                bo_h.at[idxo.at[ch]], bov.at[sl], sem))
        for c in copies:
            c.wait()

        lanes = lax.iota(jnp.int32, 16)

        def blk(b, _):
            def row(r, _):
                i = b * 16 + r
                a = (cbuf[i, pl.ds(0, 16)] * obuf[i, pl.ds(0, 16)]
                     + cbuf[i, pl.ds(16, 16)] * obuf[i, pl.ds(16, 16)])
                sbuf[r, :] = a
                return 0

            lax.fori_loop(0, 16, row, 0, unroll=True)
            acc = bcv[pl.ds(b * 16, 16)] + bov[pl.ds(b * 16, 16)]

            def col(j, acc):
                cols = jnp.full((16,), 0, jnp.int32) + j
                return acc + plsc.load_gather(sbuf, [lanes, cols])

            acc = lax.fori_loop(0, 16, col, acc, unroll=True)
            predv[pl.ds(b * 16, 16)] = acc
            return 0

        lax.fori_loop(0, BPW // 16, blk, 0)
        pltpu.sync_copy(predv, pred_h.at[pl.ds(wid * BPW, BPW)])

    return k


def _tc_loss_body(pred_ref, coocs_ref, w_ref, out_ref):
    d = pred_ref[...] - jnp.log(coocs_ref[...])
    out_ref[...] = (jnp.sum(w_ref[...] * d * d) * (1.0 / B)).reshape(1, 1)


def kernel(center, outside, coocs, weighting,
           W_center, W_outside, b_center, b_outside):
    center_r = center.reshape(NW, NCH, CH)
    outside_r = outside.reshape(NW, NCH, CH)
    bc = b_center.reshape(VOC)
    bo = b_outside.reshape(VOC)

    pred = _sc_pred_kernel()(center_r, outside_r, W_center, W_outside, bc, bo)

    loss = pl.pallas_call(
        _tc_loss_body,
        out_shape=jax.ShapeDtypeStruct((1, 1), jnp.float32),
    )(pred.reshape(128, 128), coocs.reshape(128, 128),
      weighting.reshape(128, 128))
    return loss.reshape(())
